# baseline (speedup 1.0000x reference)
import functools

import jax
import jax.numpy as jnp
from jax import lax
from jax.experimental import pallas as pl
from jax.experimental.pallas import tpu as pltpu
from jax.experimental.pallas import tpu_sc as plsc

D = 16
NC = 2
NS = 16
NW = NC * NS
IB = 128


def _sc_gather_sum(x2d, pmap, smap, emb, pemb, semb, n_rows):
    ch = n_rows // NW
    jn = ch // IB
    mesh = plsc.VectorSubcoreMesh(
        core_axis_name="c", subcore_axis_name="s", num_cores=NC,
        num_subcores=NS)

    @functools.partial(
        pl.kernel,
        out_type=jax.ShapeDtypeStruct((n_rows, D), jnp.float32),
        mesh=mesh,
        scratch_types=[
            pltpu.VMEM((ch,), jnp.int32),
            pltpu.VMEM((ch,), jnp.int32),
            pltpu.VMEM((ch,), jnp.int32),
            pltpu.VMEM((ch, D), jnp.float32),
            pltpu.VMEM((ch, D), jnp.float32),
            pltpu.SemaphoreType.DMA,
            pltpu.SemaphoreType.DMA,
        ],
        compiler_params=pltpu.CompilerParams(use_tc_tiling_on_sc=False),
    )
    def sc_kernel(x_hbm, pmap_hbm, smap_hbm, emb_hbm, pemb_hbm, semb_hbm,
                  out_hbm, xv, pmv, smv, acc, tmp, sem_rows, sem_idx):
        wid = lax.axis_index("s") * NC + lax.axis_index("c")
        pltpu.sync_copy(x_hbm.at[pl.ds(wid * ch, ch)], xv)

        main_cps = []
        for j in range(jn):
            cp = pltpu.make_async_copy(
                emb_hbm.at[xv.at[pl.ds(j * IB, IB)]],
                acc.at[pl.ds(j * IB, IB)], sem_rows)
            cp.start()
            main_cps.append(cp)
        idx_cps = []
        for j in range(jn):
            cp = pltpu.make_async_copy(
                pmap_hbm.at[xv.at[pl.ds(j * IB, IB)]],
                pmv.at[pl.ds(j * IB, IB)], sem_idx)
            cp.start()
            idx_cps.append(cp)
        for j in range(jn):
            cp = pltpu.make_async_copy(
                smap_hbm.at[xv.at[pl.ds(j * IB, IB)]],
                smv.at[pl.ds(j * IB, IB)], sem_idx)
            cp.start()
            idx_cps.append(cp)
        for cp in idx_cps:
            cp.wait()

        pre_cps = []
        for j in range(jn):
            cp = pltpu.make_async_copy(
                pemb_hbm.at[pmv.at[pl.ds(j * IB, IB)]],
                tmp.at[pl.ds(j * IB, IB)], sem_idx)
            cp.start()
            pre_cps.append(cp)
        for cp in main_cps:
            cp.wait()
        for cp in pre_cps:
            cp.wait()

        def add_body(i, carry):
            acc[i] = acc[i] + tmp[i]
            return carry
        lax.fori_loop(0, ch, add_body, 0)

        suf_cps = []
        for j in range(jn):
            cp = pltpu.make_async_copy(
                semb_hbm.at[smv.at[pl.ds(j * IB, IB)]],
                tmp.at[pl.ds(j * IB, IB)], sem_idx)
            cp.start()
            suf_cps.append(cp)
        for cp in suf_cps:
            cp.wait()
        lax.fori_loop(0, ch, add_body, 0)

        pltpu.sync_copy(acc, out_hbm.at[pl.ds(wid * ch, ch)])

    return sc_kernel(x2d, pmap, smap, emb, pemb, semb)


def _mlp_body(h_ref, w1_ref, b1_ref, w2_ref, b2_ref, out_ref):
    h = jnp.tanh(
        jnp.dot(h_ref[...], w1_ref[...], preferred_element_type=jnp.float32)
        + b1_ref[...])
    logits = (
        jnp.dot(h, w2_ref[...], preferred_element_type=jnp.float32)
        + b2_ref[...])
    m = jnp.max(logits, axis=1, keepdims=True)
    e = jnp.exp(logits - m)
    out_ref[...] = e / jnp.sum(e, axis=1, keepdims=True)


def kernel(x, embed, embed_prefix, embed_suffix, prefix_map, suffix_map,
           W1, b1, W2, b2):
    bsz, win = x.shape
    n_rows = bsz * win
    hid = W1.shape[1]
    out_dim = W2.shape[1]

    x_flat = x.reshape(n_rows)
    rows = _sc_gather_sum(x_flat, prefix_map, suffix_map, embed,
                          embed_prefix, embed_suffix, n_rows)
    h = rows.reshape(bsz, win * D)

    blk = 1024
    out = pl.pallas_call(
        _mlp_body,
        grid=(bsz // blk,),
        in_specs=[
            pl.BlockSpec((blk, win * D), lambda i: (i, 0)),
            pl.BlockSpec((win * D, hid), lambda i: (0, 0)),
            pl.BlockSpec((1, hid), lambda i: (0, 0)),
            pl.BlockSpec((hid, out_dim), lambda i: (0, 0)),
            pl.BlockSpec((1, out_dim), lambda i: (0, 0)),
        ],
        out_specs=pl.BlockSpec((blk, out_dim), lambda i: (i, 0)),
        out_shape=jax.ShapeDtypeStruct((bsz, out_dim), jnp.float32),
    )(h, W1, b1.reshape(1, hid), W2, b2.reshape(1, out_dim))
    return out


# speedup vs baseline: 1.1081x; 1.1081x over previous
import functools

import jax
import jax.numpy as jnp
from jax import lax
from jax.experimental import pallas as pl
from jax.experimental.pallas import tpu as pltpu
from jax.experimental.pallas import tpu_sc as plsc

D = 16
NC = 2
NS = 16
NW = NC * NS
IB = 128
RBLK = 4096


def _pad_rows(v):
    return -(-v // 1024) * 1024


def _cm_body(in_ref, out_ref):
    k, blk = in_ref.shape
    out_ref[...] = in_ref[...].reshape(k, blk // 128, 128)


def _to_col_major_flat(table_t):
    k, v = table_t.shape
    vp = _pad_rows(v)
    out = pl.pallas_call(
        _cm_body,
        grid=(-(-v // RBLK),),
        in_specs=[pl.BlockSpec((k, RBLK), lambda i: (0, i))],
        out_specs=pl.BlockSpec((k, RBLK // 128, 128), lambda i: (0, i, 0)),
        out_shape=jax.ShapeDtypeStruct((k, vp // 128, 128), jnp.float32),
    )(table_t)
    return out.reshape(k * vp)


def _sc_gather_sum(x_flat, pmap, smap, embf, pembf, sembf, n_rows,
                   vp_e, vp_p, vp_s):
    ch = n_rows // NW
    mesh = plsc.VectorSubcoreMesh(
        core_axis_name="c", subcore_axis_name="s", num_cores=NC,
        num_subcores=NS)

    @functools.partial(
        pl.kernel,
        out_type=jax.ShapeDtypeStruct((D, n_rows), jnp.float32),
        mesh=mesh,
        scratch_types=(
            [pltpu.VMEM((ch,), jnp.int32)] * 3
            + [pltpu.VMEM((ch,), jnp.float32)] * D
            + [pltpu.VMEM((ch,), jnp.float32)] * D
            + [pltpu.SemaphoreType.DMA, pltpu.SemaphoreType.DMA]
        ),
    )
    def sc_kernel(x_hbm, pmap_hbm, smap_hbm, emb_hbm, pemb_hbm, semb_hbm,
                  out_hbm, xv, pmv, smv, *rest):
        acc = rest[:D]
        tmp = rest[D:2 * D]
        sem_g, sem_m = rest[2 * D:]
        wid = lax.axis_index("s") * NC + lax.axis_index("c")
        base = wid * ch
        pltpu.sync_copy(x_hbm.at[pl.ds(base, ch)], xv)

        map_cps = [
            pltpu.make_async_copy(pmap_hbm.at[xv], pmv, sem_m),
            pltpu.make_async_copy(smap_hbm.at[xv], smv, sem_m),
        ]
        for cp in map_cps:
            cp.start()

        main_cps = []
        for c in range(D):
            cp = pltpu.make_async_copy(
                emb_hbm.at[pl.ds(c * vp_e, vp_e)].at[xv], acc[c], sem_g)
            cp.start()
            main_cps.append(cp)

        for cp in map_cps:
            cp.wait()

        def add_col(c):
            def body(j, carry):
                for l in range(8):
                    s = pl.ds(j * 128 + l * D, D)
                    acc[c][s] = acc[c][s] + tmp[c][s]
                return carry
            lax.fori_loop(0, ch // 128, body, 0)

        pre_cps = []
        for c in range(D):
            cp = pltpu.make_async_copy(
                pemb_hbm.at[pl.ds(c * vp_p, vp_p)].at[pmv], tmp[c], sem_m)
            cp.start()
            pre_cps.append(cp)
        for cp in main_cps:
            cp.wait()

        suf_cps = []
        for c in range(D):
            pre_cps[c].wait()
            add_col(c)
            cp = pltpu.make_async_copy(
                semb_hbm.at[pl.ds(c * vp_s, vp_s)].at[smv], tmp[c], sem_g)
            cp.start()
            suf_cps.append(cp)
        for c in range(D):
            suf_cps[c].wait()
            add_col(c)
            pltpu.sync_copy(acc[c], out_hbm.at[c, pl.ds(base, ch)])

    return sc_kernel(x_flat, pmap, smap, embf, pembf, sembf)


def _mlp_body(h_ref, w1_ref, b1_ref, w2_ref, b2_ref, out_ref):
    h = jnp.tanh(
        jnp.dot(h_ref[...], w1_ref[...], preferred_element_type=jnp.float32)
        + b1_ref[...])
    logits = (
        jnp.dot(h, w2_ref[...], preferred_element_type=jnp.float32)
        + b2_ref[...])
    m = jnp.max(logits, axis=1, keepdims=True)
    e = jnp.exp(logits - m)
    out_ref[...] = e / jnp.sum(e, axis=1, keepdims=True)


def kernel(x, embed, embed_prefix, embed_suffix, prefix_map, suffix_map,
           W1, b1, W2, b2):
    bsz, win = x.shape
    n_rows = bsz * win
    hid = W1.shape[1]
    out_dim = W2.shape[1]

    x_flat = x.reshape(n_rows)
    embf = _to_col_major_flat(embed.T)
    pembf = _to_col_major_flat(embed_prefix.T)
    sembf = _to_col_major_flat(embed_suffix.T)

    cols = _sc_gather_sum(
        x_flat, prefix_map, suffix_map, embf, pembf, sembf, n_rows,
        _pad_rows(embed.shape[0]), _pad_rows(embed_prefix.shape[0]),
        _pad_rows(embed_suffix.shape[0]))

    h = cols.reshape(D, bsz, win).transpose(1, 2, 0).reshape(bsz, win * D)

    blk = 1024
    out = pl.pallas_call(
        _mlp_body,
        grid=(bsz // blk,),
        in_specs=[
            pl.BlockSpec((blk, win * D), lambda i: (i, 0)),
            pl.BlockSpec((win * D, hid), lambda i: (0, 0)),
            pl.BlockSpec((1, hid), lambda i: (0, 0)),
            pl.BlockSpec((hid, out_dim), lambda i: (0, 0)),
            pl.BlockSpec((1, out_dim), lambda i: (0, 0)),
        ],
        out_specs=pl.BlockSpec((blk, out_dim), lambda i: (i, 0)),
        out_shape=jax.ShapeDtypeStruct((bsz, out_dim), jnp.float32),
    )(h, W1, b1.reshape(1, hid), W2, b2.reshape(1, out_dim))
    return out


# speedup vs baseline: 1.4485x; 1.3072x over previous
import functools

import jax
import jax.numpy as jnp
from jax import lax
from jax.experimental import pallas as pl
from jax.experimental.pallas import tpu as pltpu
from jax.experimental.pallas import tpu_sc as plsc

D = 16
NC = 2
NS = 16
NW = NC * NS
IB = 128
RBLK = 4096


def _pad_rows(v):
    return -(-v // 1024) * 1024


def _cm_body(in_ref, out_ref):
    k, blk = in_ref.shape
    out_ref[...] = in_ref[...].reshape(k, blk // 128, 128)


def _to_col_major_flat(table_t, blk=RBLK):
    k, v = table_t.shape
    vp = _pad_rows(v)
    out = pl.pallas_call(
        _cm_body,
        grid=(-(-v // blk),),
        in_specs=[pl.BlockSpec((k, blk), lambda i: (0, i))],
        out_specs=pl.BlockSpec((k, blk // 128, 128), lambda i: (0, i, 0)),
        out_shape=jax.ShapeDtypeStruct((k, vp // 128, 128), table_t.dtype),
    )(table_t)
    return out.reshape(k * vp)


def _sc_gather_sum(x_flat, pmap, smap, embf, pembf, sembf, n_rows,
                   vp_e, vp_p, vp_s):
    ch = n_rows // NW
    mesh = plsc.VectorSubcoreMesh(
        core_axis_name="c", subcore_axis_name="s", num_cores=NC,
        num_subcores=NS)

    @functools.partial(
        pl.kernel,
        out_type=jax.ShapeDtypeStruct((D, n_rows), jnp.float32),
        mesh=mesh,
        scratch_types=(
            [pltpu.VMEM((ch,), jnp.int32)] * 3
            + [pltpu.VMEM((ch,), jnp.float32)] * D
            + [pltpu.VMEM((ch,), jnp.float32)] * D
            + [pltpu.SemaphoreType.DMA, pltpu.SemaphoreType.DMA]
        ),
    )
    def sc_kernel(x_hbm, pmap_hbm, smap_hbm, emb_hbm, pemb_hbm, semb_hbm,
                  out_hbm, xv, pmv, smv, *rest):
        acc = rest[:D]
        tmp = rest[D:2 * D]
        sem_g, sem_m = rest[2 * D:]
        wid = lax.axis_index("s") * NC + lax.axis_index("c")
        base = wid * ch
        pltpu.sync_copy(x_hbm.at[pl.ds(base, ch)], xv)

        map_cps = [
            pltpu.make_async_copy(pmap_hbm.at[xv], pmv, sem_m),
            pltpu.make_async_copy(smap_hbm.at[xv], smv, sem_m),
        ]
        for cp in map_cps:
            cp.start()

        main_cps = []
        for c in range(D):
            cp = pltpu.make_async_copy(
                emb_hbm.at[pl.ds(c * vp_e, vp_e)].at[xv], acc[c], sem_g)
            cp.start()
            main_cps.append(cp)

        for cp in map_cps:
            cp.wait()

        def add_col(c):
            def body(j, carry):
                for l in range(8):
                    s = pl.ds(j * 128 + l * D, D)
                    acc[c][s] = acc[c][s] + tmp[c][s]
                return carry
            lax.fori_loop(0, ch // 128, body, 0)

        pre_cps = []
        for c in range(D):
            cp = pltpu.make_async_copy(
                pemb_hbm.at[pl.ds(c * vp_p, vp_p)].at[pmv], tmp[c], sem_m)
            cp.start()
            pre_cps.append(cp)
        for cp in main_cps:
            cp.wait()

        suf_cps = []
        for c in range(D):
            pre_cps[c].wait()
            add_col(c)
            cp = pltpu.make_async_copy(
                semb_hbm.at[pl.ds(c * vp_s, vp_s)].at[smv], tmp[c], sem_g)
            cp.start()
            suf_cps.append(cp)
        for c in range(D):
            suf_cps[c].wait()
            add_col(c)
            pltpu.sync_copy(acc[c], out_hbm.at[c, pl.ds(base, ch)])

    return sc_kernel(x_flat, pmap, smap, embf, pembf, sembf)


def _mlp_body(cols_ref, w1_ref, b1_ref, w2_ref, b2_ref, out_ref):
    d, win, blk = cols_ref.shape
    g = cols_ref[...].reshape(d * win, blk)
    h = jnp.tanh(
        jax.lax.dot_general(g, w1_ref[...], (((0,), (0,)), ((), ())),
                            preferred_element_type=jnp.float32)
        + b1_ref[...])
    logits = (
        jnp.dot(h, w2_ref[...], preferred_element_type=jnp.float32)
        + b2_ref[...])
    m = jnp.max(logits, axis=1, keepdims=True)
    e = jnp.exp(logits - m)
    out_ref[...] = e / jnp.sum(e, axis=1, keepdims=True)


def kernel(x, embed, embed_prefix, embed_suffix, prefix_map, suffix_map,
           W1, b1, W2, b2):
    bsz, win = x.shape
    n_rows = bsz * win
    hid = W1.shape[1]
    out_dim = W2.shape[1]

    x_wm = _to_col_major_flat(x.T, blk=bsz)
    embf = _to_col_major_flat(embed.T)
    pembf = _to_col_major_flat(embed_prefix.T)
    sembf = _to_col_major_flat(embed_suffix.T)

    cols = _sc_gather_sum(
        x_wm, prefix_map, suffix_map, embf, pembf, sembf, n_rows,
        _pad_rows(embed.shape[0]), _pad_rows(embed_prefix.shape[0]),
        _pad_rows(embed_suffix.shape[0]))

    cols3 = cols.reshape(D, win, bsz)
    w1r = W1.reshape(win, D, hid).transpose(1, 0, 2).reshape(win * D, hid)

    blk = 1024
    out = pl.pallas_call(
        _mlp_body,
        grid=(bsz // blk,),
        in_specs=[
            pl.BlockSpec((D, win, blk), lambda i: (0, 0, i)),
            pl.BlockSpec((win * D, hid), lambda i: (0, 0)),
            pl.BlockSpec((1, hid), lambda i: (0, 0)),
            pl.BlockSpec((hid, out_dim), lambda i: (0, 0)),
            pl.BlockSpec((1, out_dim), lambda i: (0, 0)),
        ],
        out_specs=pl.BlockSpec((blk, out_dim), lambda i: (i, 0)),
        out_shape=jax.ShapeDtypeStruct((bsz, out_dim), jnp.float32),
    )(cols3, w1r, b1.reshape(1, hid), W2, b2.reshape(1, out_dim))
    return out


# speedup vs baseline: 1.9463x; 1.3437x over previous
import functools

import jax
import jax.numpy as jnp
from jax import lax
from jax.experimental import pallas as pl
from jax.experimental.pallas import tpu as pltpu
from jax.experimental.pallas import tpu_sc as plsc

D = 16
NC = 2
NS = 16
NW = NC * NS
IB = 128
RBLK = 16384


def _pad_rows(v):
    return -(-v // 1024) * 1024


def _cm_body(in_ref, out_ref):
    k, blk = in_ref.shape
    out_ref[...] = in_ref[...].reshape(k, blk // 128, 128)


def _to_col_major_flat(table_t, blk=RBLK):
    k, v = table_t.shape
    vp = _pad_rows(v)
    out = pl.pallas_call(
        _cm_body,
        grid=(-(-v // blk),),
        in_specs=[pl.BlockSpec((k, blk), lambda i: (0, i))],
        out_specs=pl.BlockSpec((k, blk // 128, 128), lambda i: (0, i, 0)),
        out_shape=jax.ShapeDtypeStruct((k, vp // 128, 128), table_t.dtype),
    )(table_t)
    return out.reshape(k * vp)


def _sc_gather_sum(x_flat, pmap, smap, embf, pembf, sembf, n_rows,
                   vp_e, vp_p, vp_s):
    ch = n_rows // NW
    mesh = plsc.VectorSubcoreMesh(
        core_axis_name="c", subcore_axis_name="s", num_cores=NC,
        num_subcores=NS)

    @functools.partial(
        pl.kernel,
        out_type=jax.ShapeDtypeStruct((D, n_rows), jnp.float32),
        mesh=mesh,
        scratch_types=(
            [pltpu.VMEM((ch,), jnp.int32)] * 3
            + [pltpu.VMEM((ch,), jnp.float32)] * D
            + [pltpu.VMEM((ch,), jnp.float32)] * D
            + [pltpu.SemaphoreType.DMA, pltpu.SemaphoreType.DMA]
        ),
    )
    def sc_kernel(x_hbm, pmap_hbm, smap_hbm, emb_hbm, pemb_hbm, semb_hbm,
                  out_hbm, xv, pmv, smv, *rest):
        acc = rest[:D]
        tmp = rest[D:2 * D]
        sem_g, sem_m = rest[2 * D:]
        wid = lax.axis_index("s") * NC + lax.axis_index("c")
        base = wid * ch
        pltpu.sync_copy(x_hbm.at[pl.ds(base, ch)], xv)

        map_cps = [
            pltpu.make_async_copy(pmap_hbm.at[xv], pmv, sem_m),
            pltpu.make_async_copy(smap_hbm.at[xv], smv, sem_m),
        ]
        for cp in map_cps:
            cp.start()

        main_cps = []
        for c in range(D):
            cp = pltpu.make_async_copy(
                emb_hbm.at[pl.ds(c * vp_e, vp_e)].at[xv], acc[c], sem_g)
            cp.start()
            main_cps.append(cp)

        for cp in map_cps:
            cp.wait()

        def add_col(c):
            def body(j, carry):
                for l in range(8):
                    s = pl.ds(j * 128 + l * D, D)
                    acc[c][s] = acc[c][s] + tmp[c][s]
                return carry
            lax.fori_loop(0, ch // 128, body, 0)

        pre_cps = []
        for c in range(D):
            cp = pltpu.make_async_copy(
                pemb_hbm.at[pl.ds(c * vp_p, vp_p)].at[pmv], tmp[c], sem_m)
            cp.start()
            pre_cps.append(cp)
        for cp in main_cps:
            cp.wait()

        suf_cps = []
        for c in range(D):
            pre_cps[c].wait()
            add_col(c)
            cp = pltpu.make_async_copy(
                semb_hbm.at[pl.ds(c * vp_s, vp_s)].at[smv], tmp[c], sem_g)
            cp.start()
            suf_cps.append(cp)
        for c in range(D):
            suf_cps[c].wait()
            add_col(c)
            pltpu.sync_copy(acc[c], out_hbm.at[c, pl.ds(base, ch)])

    return sc_kernel(x_flat, pmap, smap, embf, pembf, sembf)


def _mlp_body(cols_ref, w1_ref, b1_ref, w2_ref, b2_ref, out_ref):
    d, win, blk = cols_ref.shape
    g = cols_ref[...].reshape(d * win, blk)
    h = jnp.tanh(
        jax.lax.dot_general(g, w1_ref[...], (((0,), (0,)), ((), ())),
                            preferred_element_type=jnp.float32)
        + b1_ref[...])
    logits = (
        jnp.dot(h, w2_ref[...], preferred_element_type=jnp.float32)
        + b2_ref[...])
    m = jnp.max(logits, axis=1, keepdims=True)
    e = jnp.exp(logits - m)
    out_ref[...] = e / jnp.sum(e, axis=1, keepdims=True)


def kernel(x, embed, embed_prefix, embed_suffix, prefix_map, suffix_map,
           W1, b1, W2, b2):
    bsz, win = x.shape
    n_rows = bsz * win
    hid = W1.shape[1]
    out_dim = W2.shape[1]

    x_wm = _to_col_major_flat(x.T, blk=bsz)
    embf = _to_col_major_flat(embed.T)
    pembf = _to_col_major_flat(embed_prefix.T)
    sembf = _to_col_major_flat(embed_suffix.T)

    cols = _sc_gather_sum(
        x_wm, prefix_map, suffix_map, embf, pembf, sembf, n_rows,
        _pad_rows(embed.shape[0]), _pad_rows(embed_prefix.shape[0]),
        _pad_rows(embed_suffix.shape[0]))

    cols3 = cols.reshape(D, win, bsz)
    w1r = W1.reshape(win, D, hid).transpose(1, 0, 2).reshape(win * D, hid)

    blk = 1024
    out = pl.pallas_call(
        _mlp_body,
        grid=(bsz // blk,),
        in_specs=[
            pl.BlockSpec((D, win, blk), lambda i: (0, 0, i)),
            pl.BlockSpec((win * D, hid), lambda i: (0, 0)),
            pl.BlockSpec((1, hid), lambda i: (0, 0)),
            pl.BlockSpec((hid, out_dim), lambda i: (0, 0)),
            pl.BlockSpec((1, out_dim), lambda i: (0, 0)),
        ],
        out_specs=pl.BlockSpec((blk, out_dim), lambda i: (i, 0)),
        out_shape=jax.ShapeDtypeStruct((bsz, out_dim), jnp.float32),
    )(cols3, w1r, b1.reshape(1, hid), W2, b2.reshape(1, out_dim))
    return out


# speedup vs baseline: 2.0992x; 1.0786x over previous
import functools

import jax
import jax.numpy as jnp
from jax import lax
from jax.experimental import pallas as pl
from jax.experimental.pallas import tpu as pltpu
from jax.experimental.pallas import tpu_sc as plsc

D = 16
NC = 2
NS = 16
NW = NC * NS
IB = 128
RBLK = 32768


def _pad_rows(v):
    return -(-v // 1024) * 1024


def _cm_body(in_ref, out_ref):
    k, blk = in_ref.shape
    out_ref[...] = in_ref[...].reshape(k, blk // 128, 128)


def _to_col_major_flat(table_t, blk=RBLK):
    k, v = table_t.shape
    vp = _pad_rows(v)
    out = pl.pallas_call(
        _cm_body,
        grid=(-(-v // blk),),
        in_specs=[pl.BlockSpec((k, blk), lambda i: (0, i))],
        out_specs=pl.BlockSpec((k, blk // 128, 128), lambda i: (0, i, 0)),
        out_shape=jax.ShapeDtypeStruct((k, vp // 128, 128), table_t.dtype),
    )(table_t)
    return out.reshape(k * vp)


def _sc_gather_sum(x_flat, pmap, smap, embf, pembf, sembf, n_rows,
                   vp_e, vp_p, vp_s):
    ch = n_rows // NW
    mesh = plsc.VectorSubcoreMesh(
        core_axis_name="c", subcore_axis_name="s", num_cores=NC,
        num_subcores=NS)

    @functools.partial(
        pl.kernel,
        out_type=jax.ShapeDtypeStruct((D, n_rows), jnp.float32),
        mesh=mesh,
        scratch_types=(
            [pltpu.VMEM((ch,), jnp.int32)] * 3
            + [pltpu.VMEM((ch,), jnp.float32)] * D
            + [pltpu.VMEM((ch,), jnp.float32)] * D
            + [pltpu.SemaphoreType.DMA, pltpu.SemaphoreType.DMA]
        ),
    )
    def sc_kernel(x_hbm, pmap_hbm, smap_hbm, emb_hbm, pemb_hbm, semb_hbm,
                  out_hbm, xv, pmv, smv, *rest):
        acc = rest[:D]
        tmp = rest[D:2 * D]
        sem_g, sem_m = rest[2 * D:]
        wid = lax.axis_index("s") * NC + lax.axis_index("c")
        base = wid * ch
        pltpu.sync_copy(x_hbm.at[pl.ds(base, ch)], xv)

        map_cps = [
            pltpu.make_async_copy(pmap_hbm.at[xv], pmv, sem_m),
            pltpu.make_async_copy(smap_hbm.at[xv], smv, sem_m),
        ]
        for cp in map_cps:
            cp.start()

        main_cps = []
        for c in range(D):
            cp = pltpu.make_async_copy(
                emb_hbm.at[pl.ds(c * vp_e, vp_e)].at[xv], acc[c], sem_g)
            cp.start()
            main_cps.append(cp)

        for cp in map_cps:
            cp.wait()

        def add_col(c):
            def body(j, carry):
                for l in range(8):
                    s = pl.ds(j * 128 + l * D, D)
                    acc[c][s] = acc[c][s] + tmp[c][s]
                return carry
            lax.fori_loop(0, ch // 128, body, 0)

        pre_cps = []
        for c in range(D):
            cp = pltpu.make_async_copy(
                pemb_hbm.at[pl.ds(c * vp_p, vp_p)].at[pmv], tmp[c], sem_m)
            cp.start()
            pre_cps.append(cp)
        for cp in main_cps:
            cp.wait()

        suf_cps = []
        for c in range(D):
            pre_cps[c].wait()
            add_col(c)
            cp = pltpu.make_async_copy(
                semb_hbm.at[pl.ds(c * vp_s, vp_s)].at[smv], tmp[c], sem_g)
            cp.start()
            suf_cps.append(cp)
        for c in range(D):
            suf_cps[c].wait()
            add_col(c)
            pltpu.sync_copy(acc[c], out_hbm.at[c, pl.ds(base, ch)])

    return sc_kernel(x_flat, pmap, smap, embf, pembf, sembf)


def _mlp_body(cols_ref, w1_ref, b1_ref, w2_ref, b2_ref, out_ref):
    d, win, blk = cols_ref.shape
    g = cols_ref[...].reshape(d * win, blk)
    h = jnp.tanh(
        jax.lax.dot_general(g, w1_ref[...], (((0,), (0,)), ((), ())),
                            preferred_element_type=jnp.float32)
        + b1_ref[...])
    logits = (
        jnp.dot(h, w2_ref[...], preferred_element_type=jnp.float32)
        + b2_ref[...])
    m = jnp.max(logits, axis=1, keepdims=True)
    e = jnp.exp(logits - m)
    out_ref[...] = e / jnp.sum(e, axis=1, keepdims=True)


def kernel(x, embed, embed_prefix, embed_suffix, prefix_map, suffix_map,
           W1, b1, W2, b2):
    bsz, win = x.shape
    n_rows = bsz * win
    hid = W1.shape[1]
    out_dim = W2.shape[1]

    x_wm = _to_col_major_flat(x.T, blk=bsz)
    embf = _to_col_major_flat(embed.T)
    pembf = _to_col_major_flat(embed_prefix.T)
    sembf = _to_col_major_flat(embed_suffix.T)

    cols = _sc_gather_sum(
        x_wm, prefix_map, suffix_map, embf, pembf, sembf, n_rows,
        _pad_rows(embed.shape[0]), _pad_rows(embed_prefix.shape[0]),
        _pad_rows(embed_suffix.shape[0]))

    cols3 = cols.reshape(D, win, bsz)
    w1r = W1.reshape(win, D, hid).transpose(1, 0, 2).reshape(win * D, hid)

    blk = 1024
    out = pl.pallas_call(
        _mlp_body,
        grid=(bsz // blk,),
        in_specs=[
            pl.BlockSpec((D, win, blk), lambda i: (0, 0, i)),
            pl.BlockSpec((win * D, hid), lambda i: (0, 0)),
            pl.BlockSpec((1, hid), lambda i: (0, 0)),
            pl.BlockSpec((hid, out_dim), lambda i: (0, 0)),
            pl.BlockSpec((1, out_dim), lambda i: (0, 0)),
        ],
        out_specs=pl.BlockSpec((blk, out_dim), lambda i: (i, 0)),
        out_shape=jax.ShapeDtypeStruct((bsz, out_dim), jnp.float32),
    )(cols3, w1r, b1.reshape(1, hid), W2, b2.reshape(1, out_dim))
    return out
